# baseline (speedup 1.0000x reference)
import functools

import jax
import jax.numpy as jnp
from jax import lax
from jax.experimental import pallas as pl
from jax.experimental.pallas import tpu as pltpu
from jax.experimental.pallas import tpu_sc as plsc

_N = 10000
_DN = 128
_DE = 16
_NC = 2
_NS = 16
_NPAD = 10240
_EPAD = 323584
_NBUF = 5
_KOFF = 4
_EA_NB = 3
_L = 64
_GROUPS_PER_WORKER = _EPAD // _L // (_NC * _NS)
_CHUNKS_PER_WORKER = _GROUPS_PER_WORKER
_ROWS_PER_SUB = _NPAD // _NS


def _sc_aggregate(x, srcg, dstg, z_wide):
    mesh = plsc.VectorSubcoreMesh(core_axis_name="c", subcore_axis_name="s")
    out_type = jax.ShapeDtypeStruct((_NC, _NPAD, _DN), jnp.float32)
    scratch = [pltpu.VMEM_SHARED((_NPAD, _DN), jnp.float32)]

    def body(x_hbm, srcg_hbm, dstg_hbm, zw_hbm, p_hbm, acc):
        cid = lax.axis_index("c")
        sid = lax.axis_index("s")
        row0 = sid * _ROWS_PER_SUB

        pltpu.sync_copy(zw_hbm.at[pl.ds(row0, _ROWS_PER_SUB)],
                        acc.at[pl.ds(row0, _ROWS_PER_SUB)])
        plsc.subcore_barrier()

        base_g = (cid * _NS + sid) * _GROUPS_PER_WORKER

        def edge_loop(idx_s, idx_d, rows, *sems):
            sem_i = sems[0:_NBUF]
            sem_g = sems[_NBUF:2 * _NBUF]
            sem_s = sems[2 * _NBUF:3 * _NBUF]

            def rows_at(b):
                return rows.at[pl.ds(b * _L, _L)]

            def issue_idx(c, b):
                pltpu.async_copy(srcg_hbm.at[base_g + c], idx_s.at[b],
                                 sem_i[b])
                pltpu.async_copy(dstg_hbm.at[base_g + c], idx_d.at[b],
                                 sem_i[b])

            def wait_idx(c, b):
                pltpu.make_async_copy(srcg_hbm.at[base_g + c], idx_s.at[b],
                                      sem_i[b]).wait()
                pltpu.make_async_copy(dstg_hbm.at[base_g + c], idx_d.at[b],
                                      sem_i[b]).wait()

            def issue_gather(b):
                pltpu.async_copy(x_hbm.at[idx_s.at[b]], rows_at(b), sem_g[b])

            def wait_gather(b):
                pltpu.make_async_copy(x_hbm.at[idx_s.at[b]], rows_at(b),
                                      sem_g[b]).wait()

            def issue_scatter(b):
                pltpu.async_copy(rows_at(b), acc.at[idx_d.at[b]], sem_s[b],
                                 add=True)

            def wait_scatter(b):
                pltpu.make_async_copy(rows_at(b), acc.at[idx_d.at[b]],
                                      sem_s[b]).wait()

            nck = _CHUNKS_PER_WORKER

            @pl.loop(0, nck + _NBUF, step=_NBUF)
            def _(c0):
                for b in range(_NBUF):
                    c = c0 + b

                    @pl.when(c < nck)
                    def _():
                        @pl.when(c >= _NBUF)
                        def _():
                            wait_scatter(b)
                        issue_idx(c, b)

                    cg = c - 1
                    bg = (b - 1) % _NBUF

                    @pl.when((cg >= 0) & (cg < nck))
                    def _():
                        wait_idx(cg, bg)
                        issue_gather(bg)

                    cs = c - _KOFF
                    bs = (b - _KOFF) % _NBUF

                    @pl.when((cs >= 0) & (cs < nck))
                    def _():
                        wait_gather(bs)
                        issue_scatter(bs)

            for b in range(_NBUF):
                wait_scatter(b)

        pl.run_scoped(edge_loop,
                      pltpu.VMEM((_NBUF, _L), jnp.int32),
                      pltpu.VMEM((_NBUF, _L), jnp.int32),
                      pltpu.VMEM((_NBUF * _L, _DN), jnp.float32),
                      *([pltpu.SemaphoreType.DMA] * (3 * _NBUF)))

        plsc.subcore_barrier()
        pltpu.sync_copy(acc.at[pl.ds(row0, _ROWS_PER_SUB)],
                        p_hbm.at[cid, pl.ds(row0, _ROWS_PER_SUB)])

    k = pl.kernel(body, out_type=out_type, mesh=mesh, scratch_types=scratch)
    return k(x, srcg, dstg, z_wide)


def _sc_ea_aggregate(eaf, dstg, z_wide):
    mesh = plsc.VectorSubcoreMesh(core_axis_name="c", subcore_axis_name="s")
    out_type = jax.ShapeDtypeStruct((_NC, _NPAD, _DN), jnp.float32)
    scratch = [pltpu.VMEM_SHARED((_NPAD, _DN), jnp.float32)]

    def body(eaf_hbm, dstg_hbm, zw_hbm, pe_hbm, acc):
        cid = lax.axis_index("c")
        sid = lax.axis_index("s")
        row0 = sid * _ROWS_PER_SUB

        pltpu.sync_copy(zw_hbm.at[pl.ds(row0, _ROWS_PER_SUB)],
                        acc.at[pl.ds(row0, _ROWS_PER_SUB)])
        plsc.subcore_barrier()

        base_g = (cid * _NS + sid) * _GROUPS_PER_WORKER

        def edge_loop(idx_d, packed, wide, *sems):
            sem_i = sems[0:_EA_NB]
            sem_s = sems[_EA_NB:2 * _EA_NB]

            def wide_at(b):
                return wide.at[pl.ds(b * _L, _L)]

            nfl = _L * _DE

            def issue_loads(c, b):
                pltpu.async_copy(dstg_hbm.at[base_g + c], idx_d.at[b],
                                 sem_i[b])
                pltpu.async_copy(eaf_hbm.at[pl.ds((base_g + c) * nfl, nfl)],
                                 packed.at[pl.ds(b * nfl, nfl)], sem_i[b])

            def wait_loads(c, b):
                pltpu.make_async_copy(dstg_hbm.at[base_g + c], idx_d.at[b],
                                      sem_i[b]).wait()
                pltpu.make_async_copy(
                    eaf_hbm.at[pl.ds((base_g + c) * nfl, nfl)],
                    packed.at[pl.ds(b * nfl, nfl)], sem_i[b]).wait()

            def issue_scatter(b):
                pltpu.async_copy(wide_at(b), acc.at[idx_d.at[b]], sem_s[b],
                                 add=True)

            def wait_scatter(b):
                pltpu.make_async_copy(wide_at(b), acc.at[idx_d.at[b]],
                                      sem_s[b]).wait()

            @pl.loop(0, _EA_NB * _L)
            def _(r):
                @pl.loop(16, _DN, step=16)
                def _(j):
                    wide[r, pl.ds(j, 16)] = jnp.zeros((16,), jnp.float32)

            nck = _CHUNKS_PER_WORKER

            @pl.loop(0, nck + _EA_NB, step=_EA_NB)
            def _(c0):
                for b in range(_EA_NB):
                    c = c0 + b

                    @pl.when(c < nck)
                    def _():
                        @pl.when(c >= _EA_NB)
                        def _():
                            wait_scatter(b)
                        issue_loads(c, b)

                    ce = c - 1
                    be = (b - 1) % _EA_NB

                    @pl.when((ce >= 0) & (ce < nck))
                    def _():
                        wait_loads(ce, be)
                        @pl.loop(0, _L)
                        def _(i):
                            wide[be * _L + i, pl.ds(0, 16)] = (
                                packed[pl.ds(be * nfl + i * _DE, 16)])
                        issue_scatter(be)

            for b in range(_EA_NB):
                wait_scatter(b)

        pl.run_scoped(edge_loop,
                      pltpu.VMEM((_EA_NB, _L), jnp.int32),
                      pltpu.VMEM((_EA_NB * _L * _DE,), jnp.float32),
                      pltpu.VMEM((_EA_NB * _L, _DN), jnp.float32),
                      *([pltpu.SemaphoreType.DMA] * (2 * _EA_NB)))

        plsc.subcore_barrier()
        pltpu.sync_copy(acc.at[pl.ds(row0, _ROWS_PER_SUB)],
                        pe_hbm.at[cid, pl.ds(row0, _ROWS_PER_SUB)])

    k = pl.kernel(body, out_type=out_type, mesh=mesh, scratch_types=scratch)
    return k(eaf, dstg, z_wide)


def _tc_combine(p, eap, W, b, relu):
    blk = 1024

    def body(p_ref, e_ref, w_ref, b_ref, o_ref):
        a = p_ref[0] + p_ref[1]
        e = (e_ref[0] + e_ref[1])[:, :_DE]
        r = (jnp.dot(a, w_ref[0:_DN, :], preferred_element_type=jnp.float32)
             + jnp.dot(e, w_ref[_DN:, :], preferred_element_type=jnp.float32)
             + b_ref[0])
        o_ref[...] = jnp.maximum(r, 0.0) if relu else r

    return pl.pallas_call(
        body,
        grid=(_NPAD // blk,),
        in_specs=[
            pl.BlockSpec((_NC, blk, _DN), lambda i: (0, i, 0)),
            pl.BlockSpec((_NC, blk, _DN), lambda i: (0, i, 0)),
            pl.BlockSpec((_DN + _DE, _DN), lambda i: (0, 0)),
            pl.BlockSpec((1, _DN), lambda i: (0, 0)),
        ],
        out_specs=pl.BlockSpec((blk, _DN), lambda i: (i, 0)),
        out_shape=jax.ShapeDtypeStruct((_NPAD, _DN), jnp.float32),
    )(p, eap, W, b.reshape(1, _DN))


def kernel(x, edge_index, edge_attr, W1, b1, W2, b2):
    e = edge_index.shape[1]
    pad = _EPAD - e
    src = edge_index[0].astype(jnp.int32)
    dst = edge_index[1].astype(jnp.int32)
    src_p = jnp.concatenate([src, jnp.zeros((pad,), jnp.int32)])
    dst_p = jnp.concatenate([dst, jnp.full((pad,), _NPAD - 1, jnp.int32)])
    srcg = src_p.reshape(_EPAD // _L, _L)
    dstg = dst_p.reshape(_EPAD // _L, _L)
    ea_p = jnp.concatenate([edge_attr, jnp.zeros((pad, _DE), edge_attr.dtype)])
    eaf = ea_p.reshape(_EPAD * _DE)

    z_wide = jnp.zeros((_NPAD, _DN), jnp.float32)
    eap = _sc_ea_aggregate(eaf, dstg, z_wide)
    p1 = _sc_aggregate(x, srcg, dstg, z_wide)
    h = _tc_combine(p1, eap, W1, b1, relu=True)
    q = _sc_aggregate(h, srcg, dstg, z_wide)
    out_p = _tc_combine(q, eap, W2, b2, relu=False)
    return out_p[:_N]


# speedup vs baseline: 1.4508x; 1.4508x over previous
import jax
import jax.numpy as jnp
from jax import lax
from jax.experimental import pallas as pl
from jax.experimental.pallas import tpu as pltpu
from jax.experimental.pallas import tpu_sc as plsc

_N = 10000
_DN = 128
_DE = 16
_NC = 2
_NS = 16
_NPAD = 10240
_EPAD = 321024
_NBUF = 4
_KOFF = 3
_L = 88
_CHUNKS = _EPAD // _L // (_NC * _NS)
_ROWS_PER_SUB = _NPAD // _NS


def _sc_aggregate(x, sdg, z_wide):
    mesh = plsc.VectorSubcoreMesh(core_axis_name="c", subcore_axis_name="s")
    out_type = jax.ShapeDtypeStruct((_NC, _NPAD, _DN), jnp.float32)
    scratch = [pltpu.VMEM_SHARED((_NPAD, _DN), jnp.float32)]

    def body(x_hbm, sdg_hbm, zw_hbm, p_hbm, acc):
        cid = lax.axis_index("c")
        sid = lax.axis_index("s")
        row0 = sid * _ROWS_PER_SUB

        pltpu.sync_copy(zw_hbm.at[pl.ds(row0, _ROWS_PER_SUB)],
                        acc.at[pl.ds(row0, _ROWS_PER_SUB)])
        plsc.subcore_barrier()

        base_g = (cid * _NS + sid) * _CHUNKS

        def edge_loop(idxb, rows, *sems):
            sem_i = sems[0:_NBUF]
            sem_g = sems[_NBUF:2 * _NBUF]
            sem_s = sems[2 * _NBUF:3 * _NBUF]

            def rows_at(b):
                return rows.at[pl.ds(b * _L, _L)]

            def issue_idx(c, b):
                pltpu.async_copy(sdg_hbm.at[base_g + c], idxb.at[b], sem_i[b])

            def wait_idx(c, b):
                pltpu.make_async_copy(sdg_hbm.at[base_g + c], idxb.at[b],
                                      sem_i[b]).wait()

            def issue_gather(b):
                pltpu.async_copy(x_hbm.at[idxb.at[b, 0]], rows_at(b),
                                 sem_g[b])

            def wait_gather(b):
                pltpu.make_async_copy(x_hbm.at[idxb.at[b, 0]], rows_at(b),
                                      sem_g[b]).wait()

            def issue_scatter(b):
                pltpu.async_copy(rows_at(b), acc.at[idxb.at[b, 1]], sem_s[b],
                                 add=True)

            def wait_scatter(b):
                pltpu.make_async_copy(rows_at(b), acc.at[idxb.at[b, 1]],
                                      sem_s[b]).wait()

            @pl.loop(0, _CHUNKS + _NBUF, step=_NBUF)
            def _(c0):
                for b in range(_NBUF):
                    c = c0 + b

                    @pl.when(c < _CHUNKS)
                    def _():
                        @pl.when(c >= _NBUF)
                        def _():
                            wait_scatter(b)
                        issue_idx(c, b)

                    cg = c - 1
                    bg = (b - 1) % _NBUF

                    @pl.when((cg >= 0) & (cg < _CHUNKS))
                    def _():
                        wait_idx(cg, bg)
                        issue_gather(bg)

                    cs = c - _KOFF
                    bs = (b - _KOFF) % _NBUF

                    @pl.when((cs >= 0) & (cs < _CHUNKS))
                    def _():
                        wait_gather(bs)
                        issue_scatter(bs)

            for b in range(_NBUF):
                wait_scatter(b)

        pl.run_scoped(edge_loop,
                      pltpu.VMEM((_NBUF, 2, _L), jnp.int32),
                      pltpu.VMEM((_NBUF * _L, _DN), jnp.float32),
                      *([pltpu.SemaphoreType.DMA] * (3 * _NBUF)))

        plsc.subcore_barrier()
        pltpu.sync_copy(acc.at[pl.ds(row0, _ROWS_PER_SUB)],
                        p_hbm.at[cid, pl.ds(row0, _ROWS_PER_SUB)])

    k = pl.kernel(body, out_type=out_type, mesh=mesh, scratch_types=scratch)
    return k(x, sdg, z_wide)


def _tc_combine(p, eap, W, b, relu):
    blk = 1024

    def body(p_ref, e_ref, w_ref, b_ref, o_ref):
        a = p_ref[0] + p_ref[1]
        e = (e_ref[0] + e_ref[1])[:, :_DE]
        r = (jnp.dot(a, w_ref[0:_DN, :], preferred_element_type=jnp.float32)
             + jnp.dot(e, w_ref[_DN:, :], preferred_element_type=jnp.float32)
             + b_ref[0])
        o_ref[...] = jnp.maximum(r, 0.0) if relu else r

    return pl.pallas_call(
        body,
        grid=(_NPAD // blk,),
        in_specs=[
            pl.BlockSpec((_NC, blk, _DN), lambda i: (0, i, 0)),
            pl.BlockSpec((_NC, blk, _DN), lambda i: (0, i, 0)),
            pl.BlockSpec((_DN + _DE, _DN), lambda i: (0, 0)),
            pl.BlockSpec((1, _DN), lambda i: (0, 0)),
        ],
        out_specs=pl.BlockSpec((blk, _DN), lambda i: (i, 0)),
        out_shape=jax.ShapeDtypeStruct((_NPAD, _DN), jnp.float32),
    )(p, eap, W, b.reshape(1, _DN))


def kernel(x, edge_index, edge_attr, W1, b1, W2, b2):
    e = edge_index.shape[1]
    pad = _EPAD - e
    src = edge_index[0].astype(jnp.int32)
    dst = edge_index[1].astype(jnp.int32)
    src_p = jnp.concatenate([src, jnp.zeros((pad,), jnp.int32)])
    dst_p = jnp.concatenate([dst, jnp.full((pad,), _NPAD - 1, jnp.int32)])
    g = _EPAD // _L
    dst_g = dst_p.reshape(g, 1, _L)
    sdg = jnp.concatenate([src_p.reshape(g, 1, _L), dst_g], axis=1)
    ea_wide = jnp.zeros((_EPAD, _DN), jnp.float32)
    ea_wide = lax.dynamic_update_slice(ea_wide, edge_attr, (0, 0))
    iota_g = jnp.arange(_EPAD, dtype=jnp.int32).reshape(g, 1, _L)
    sdg_ea = jnp.concatenate([iota_g, dst_g], axis=1)

    z_wide = jnp.zeros((_NPAD, _DN), jnp.float32)
    p1 = _sc_aggregate(x, sdg, z_wide)
    eap = _sc_aggregate(ea_wide, sdg_ea, z_wide)
    h = _tc_combine(p1, eap, W1, b1, relu=True)
    q = _sc_aggregate(h, sdg, z_wide)
    out_p = _tc_combine(q, eap, W2, b2, relu=False)
    return out_p[:_N]
